# baseline (speedup 1.0000x reference)
import functools

import jax
import jax.numpy as jnp
from jax import lax
from jax.experimental import pallas as pl
from jax.experimental.pallas import tpu as pltpu
from jax.experimental.pallas import tpu_sc as plsc

NC = 2
NS = 16
NW = NC * NS
C = 125


NBUF = 4


def _sc_edge_scatter(ga, gb, src_l, dst_l, n_pad, hw, n_chunks):
    rps = n_pad // NS
    assert n_chunks % NBUF == 0
    mesh = plsc.VectorSubcoreMesh(core_axis_name="c", subcore_axis_name="s")

    @functools.partial(
        pl.kernel,
        mesh=mesh,
        compiler_params=pltpu.CompilerParams(use_tc_tiling_on_sc=False),
        out_type=(jax.ShapeDtypeStruct((n_pad, hw), jnp.float32),
                  jax.ShapeDtypeStruct((n_pad, hw), jnp.float32)),
        scratch_types=[
            pltpu.VMEM((n_chunks, C), jnp.int32),
            pltpu.VMEM((n_chunks, C), jnp.int32),
        ]
        + [pltpu.VMEM((C, hw), jnp.float32) for _ in range(NBUF)]
        + [
            pltpu.VMEM_SHARED((n_pad, hw), jnp.float32),
        ]
        + [pltpu.SemaphoreType.DMA for _ in range(2 * NBUF)],
    )
    def k(ga_h, gb_h, src_h, dst_h, oa_h, ob_h, sall, dall, *rest):
        rows = rest[:NBUF]
        acc = rest[NBUF]
        gsem = rest[NBUF + 1:NBUF + 1 + NBUF]
        ssem = rest[NBUF + 1 + NBUF:]
        cid = lax.axis_index("c")
        sid = lax.axis_index("s")
        sl = pl.ds(sid * rps, rps)
        pltpu.sync_copy(src_h.at[sid], sall)
        pltpu.sync_copy(dst_h.at[sid], dall)

        def run(g_h, out_h):
            pltpu.sync_copy(g_h.at[sl], acc.at[sl])
            plsc.subcore_barrier()

            def gath(j, b):
                return pltpu.async_copy(g_h.at[sall.at[j]], rows[b],
                                        gsem[b])

            gath(0, 0)
            gath(1, 1)

            def body(i, carry):
                j0 = i * NBUF
                for u in range(NBUF):
                    j = j0 + u
                    b = u
                    bn = (u + 2) % NBUF
                    pltpu.make_async_copy(g_h.at[sall.at[j]], rows[b],
                                          gsem[b]).wait()
                    pltpu.async_copy(rows[b], acc.at[dall.at[j]],
                                     ssem[b], add=True)
                    jp = j + 2 - NBUF
                    @pl.when(jp >= 0)
                    def _():
                        pltpu.make_async_copy(rows[bn],
                                              acc.at[dall.at[jp]],
                                              ssem[bn]).wait()
                    @pl.when(j + 2 < n_chunks)
                    def _():
                        gath(j + 2, bn)
                return carry

            lax.fori_loop(0, n_chunks // NBUF, body, 0, unroll=False)
            for j in (n_chunks - 2, n_chunks - 1):
                b = j % NBUF
                pltpu.make_async_copy(rows[b], acc.at[dall.at[j]],
                                      ssem[b]).wait()
            plsc.subcore_barrier()
            pltpu.sync_copy(acc.at[sl], out_h.at[sl])

        @pl.when(cid == 0)
        def _():
            run(ga_h, oa_h)

        @pl.when(cid == 1)
        def _():
            run(gb_h, ob_h)

    return k(ga, gb, src_l, dst_l)


def _sc_degree(dst_l, n_pad, n_chunks):
    rps = n_pad // NS
    ZB = 128
    assert rps % ZB == 0 and n_chunks % NBUF == 0
    mesh = plsc.VectorSubcoreMesh(core_axis_name="c", subcore_axis_name="s")

    @functools.partial(
        pl.kernel,
        mesh=mesh,
        compiler_params=pltpu.CompilerParams(use_tc_tiling_on_sc=False),
        out_type=jax.ShapeDtypeStruct((n_pad,), jnp.float32),
        scratch_types=[
            pltpu.VMEM((n_chunks, C), jnp.int32),
            pltpu.VMEM((ZB,), jnp.float32),
            pltpu.VMEM((ZB,), jnp.float32),
            pltpu.VMEM_SHARED((n_pad,), jnp.float32),
        ]
        + [pltpu.SemaphoreType.DMA for _ in range(NBUF)],
    )
    def k(dst_h, out_h, dall, ones_v, zero_v, acc, *sems):
        cid = lax.axis_index("c")
        sid = lax.axis_index("s")

        @pl.when(cid == 0)
        def _():
            for i in range(ZB // 16):
                ones_v[pl.ds(i * 16, 16)] = jnp.ones((16,), jnp.float32)
                zero_v[pl.ds(i * 16, 16)] = jnp.zeros((16,), jnp.float32)
            pltpu.sync_copy(dst_h.at[sid], dall)
            for t in range(rps // ZB):
                pltpu.sync_copy(zero_v,
                                acc.at[pl.ds(sid * rps + t * ZB, ZB)])
            plsc.subcore_barrier()
            ones_c = ones_v.at[pl.ds(0, C)]

            def body(i, carry):
                for u in range(NBUF):
                    j = i * NBUF + u
                    @pl.when(i > 0)
                    def _():
                        pltpu.make_async_copy(ones_c,
                                              acc.at[dall.at[j - NBUF]],
                                              sems[u]).wait()
                    pltpu.async_copy(ones_c, acc.at[dall.at[j]], sems[u],
                                     add=True)
                return carry

            lax.fori_loop(0, n_chunks // NBUF, body, 0, unroll=False)
            for u in range(NBUF):
                j = n_chunks - NBUF + u
                pltpu.make_async_copy(ones_c, acc.at[dall.at[j]],
                                      sems[u]).wait()
            plsc.subcore_barrier()
            pltpu.sync_copy(acc.at[pl.ds(sid * rps, rps)],
                            out_h.at[pl.ds(sid * rps, rps)])

    return k(dst_l)


def _dinv(deg_ref):
    return lax.rsqrt(deg_ref[0, :] + 1.0)[:, None]


def kernel(x, edge_index, batch, W1, b1, W2, b2, W3, b3, Wh, bh, Wo, bo):
    N, D = x.shape
    E = edge_index.shape[1]
    H = W1.shape[1]
    H2 = W2.shape[1]
    H3 = W3.shape[1]
    OUT = Wo.shape[1]

    n_pad = ((N + 2047) // 2048) * 2048
    pad_rows = n_pad - N

    Hh = H // 2
    assert E % (NS * C * NBUF) == 0, "edge count must tile evenly"
    n_chunks = E // (NS * C)

    src_l = edge_index[0].reshape(NS, n_chunks, C)
    dst_l = edge_index[1].reshape(NS, n_chunks, C)
    x_pad = jnp.pad(x, ((0, pad_rows), (0, 0)))
    batch2d = jnp.pad(batch, (0, pad_rows), constant_values=64)[None, :]
    b1r, b2r, b3r = b1[None, :], b2[None, :], b3[None, :]
    bhr, bor = bh[None, :], bo[None, :]

    R = n_pad // NS
    T = NS

    deg = _sc_degree(dst_l, n_pad, n_chunks)
    deg2d = deg[None, :]

    def _b_body(x_ref, w_ref, deg_ref, oa_ref, ob_ref):
        h = jnp.dot(x_ref[...], w_ref[...], preferred_element_type=jnp.float32)
        g = h * _dinv(deg_ref)
        oa_ref[...] = g[:, :Hh]
        ob_ref[...] = g[:, Hh:]

    g1a, g1b = pl.pallas_call(
        _b_body,
        grid=(T,),
        in_specs=[pl.BlockSpec((R, D), lambda i: (i, 0)),
                  pl.BlockSpec((D, H), lambda i: (0, 0)),
                  pl.BlockSpec((1, R), lambda i: (0, i))],
        out_specs=[pl.BlockSpec((R, Hh), lambda i: (i, 0)),
                   pl.BlockSpec((R, Hh), lambda i: (i, 0))],
        out_shape=[jax.ShapeDtypeStruct((n_pad, Hh), jnp.float32),
                   jax.ShapeDtypeStruct((n_pad, Hh), jnp.float32)],
    )(x_pad, W1, deg2d)

    o1a, o1b = _sc_edge_scatter(g1a, g1b, src_l, dst_l, n_pad, Hh, n_chunks)

    def _c_body(oa_ref, ob_ref, deg_ref, b_ref, na_ref, nb_ref):
        dinv = _dinv(deg_ref)
        agg = dinv * jnp.concatenate([oa_ref[...], ob_ref[...]], axis=1)
        g2t = dinv * jnp.maximum(agg + b_ref[...], 0.0)
        na_ref[...] = g2t[:, :Hh]
        nb_ref[...] = g2t[:, Hh:]

    g2a, g2b = pl.pallas_call(
        _c_body,
        grid=(T,),
        in_specs=[pl.BlockSpec((R, Hh), lambda i: (i, 0)),
                  pl.BlockSpec((R, Hh), lambda i: (i, 0)),
                  pl.BlockSpec((1, R), lambda i: (0, i)),
                  pl.BlockSpec((1, H), lambda i: (0, 0))],
        out_specs=[pl.BlockSpec((R, Hh), lambda i: (i, 0)),
                   pl.BlockSpec((R, Hh), lambda i: (i, 0))],
        out_shape=[jax.ShapeDtypeStruct((n_pad, Hh), jnp.float32),
                   jax.ShapeDtypeStruct((n_pad, Hh), jnp.float32)],
    )(o1a, o1b, deg2d, b1r)

    o2a, o2b = _sc_edge_scatter(g2a, g2b, src_l, dst_l, n_pad, Hh, n_chunks)

    def _d_body(oa_ref, ob_ref, deg_ref, w_ref, b_ref, na_ref, nb_ref):
        dinv = _dinv(deg_ref)
        a = dinv * jnp.concatenate([oa_ref[...], ob_ref[...]], axis=1)
        c2 = jnp.maximum(
            jnp.dot(a, w_ref[...], preferred_element_type=jnp.float32)
            + b_ref[...], 0.0)
        g3t = dinv * c2
        na_ref[...] = g3t[:, :H]
        nb_ref[...] = g3t[:, H:]

    g3a, g3b = pl.pallas_call(
        _d_body,
        grid=(T,),
        in_specs=[pl.BlockSpec((R, Hh), lambda i: (i, 0)),
                  pl.BlockSpec((R, Hh), lambda i: (i, 0)),
                  pl.BlockSpec((1, R), lambda i: (0, i)),
                  pl.BlockSpec((H, H2), lambda i: (0, 0)),
                  pl.BlockSpec((1, H2), lambda i: (0, 0))],
        out_specs=[pl.BlockSpec((R, H), lambda i: (i, 0)),
                   pl.BlockSpec((R, H), lambda i: (i, 0))],
        out_shape=[jax.ShapeDtypeStruct((n_pad, H), jnp.float32),
                   jax.ShapeDtypeStruct((n_pad, H), jnp.float32)],
    )(o2a, o2b, deg2d, W2, b2r)

    o3a, o3b = _sc_edge_scatter(g3a, g3b, src_l, dst_l, n_pad, H, n_chunks)

    GR = 64

    def _e_body(oa_ref, ob_ref, deg_ref, w_ref, b_ref,
                batch_ref, sums_ref, cnts_ref):
        i = pl.program_id(0)

        @pl.when(i == 0)
        def _():
            sums_ref[...] = jnp.zeros_like(sums_ref)
            cnts_ref[...] = jnp.zeros_like(cnts_ref)

        dinv = _dinv(deg_ref)
        a = dinv * jnp.concatenate([oa_ref[...], ob_ref[...]], axis=1)
        c3 = jnp.maximum(
            jnp.dot(a, w_ref[...], preferred_element_type=jnp.float32)
            + b_ref[...], 0.0)
        onehot = (lax.broadcasted_iota(jnp.int32, (GR, R), 0)
                  == batch_ref[...]).astype(jnp.float32)
        sums_ref[...] += jnp.dot(onehot, c3,
                                 preferred_element_type=jnp.float32)
        cnts_ref[...] = cnts_ref[...] + jnp.sum(onehot, axis=1, keepdims=True)

    sums, cnts = pl.pallas_call(
        _e_body,
        grid=(T,),
        in_specs=[pl.BlockSpec((R, H), lambda i: (i, 0)),
                  pl.BlockSpec((R, H), lambda i: (i, 0)),
                  pl.BlockSpec((1, R), lambda i: (0, i)),
                  pl.BlockSpec((H2, H3), lambda i: (0, 0)),
                  pl.BlockSpec((1, H3), lambda i: (0, 0)),
                  pl.BlockSpec((1, R), lambda i: (0, i))],
        out_specs=[pl.BlockSpec((GR, H3), lambda i: (0, 0)),
                   pl.BlockSpec((GR, 128), lambda i: (0, 0))],
        out_shape=[jax.ShapeDtypeStruct((GR, H3), jnp.float32),
                   jax.ShapeDtypeStruct((GR, 128), jnp.float32)],
    )(o3a, o3b, deg2d, W3, b3r, batch2d)

    def _f_body(sums_ref, cnts_ref, wh_ref, bh_ref, wo_ref, bo_ref, o_ref):
        cnt = cnts_ref[:, 0:1]
        pooled = sums_ref[...] / jnp.maximum(cnt, 1.0)
        hid = jnp.maximum(
            jnp.dot(pooled, wh_ref[...], preferred_element_type=jnp.float32)
            + bh_ref[...], 0.0)
        logits = jnp.dot(hid, wo_ref[...],
                         preferred_element_type=jnp.float32) + bo_ref[...]
        o_ref[...] = jax.nn.sigmoid(logits)

    out = pl.pallas_call(
        _f_body,
        out_shape=jax.ShapeDtypeStruct((GR, OUT), jnp.float32),
    )(sums, cnts, Wh, bhr, Wo, bor)

    return out


# speedup vs baseline: 1.1483x; 1.1483x over previous
import functools

import jax
import jax.numpy as jnp
from jax import lax
from jax.experimental import pallas as pl
from jax.experimental.pallas import tpu as pltpu
from jax.experimental.pallas import tpu_sc as plsc

NC = 2
NS = 16
NW = NC * NS
C = 128
PF = 3


NBUF = 5


def _sc_edge_scatter(ga, gb, src_l, dst_l, n_pad, hw, n_chunks):
    rps = n_pad // NS
    assert n_chunks % NBUF == 0
    mesh = plsc.VectorSubcoreMesh(core_axis_name="c", subcore_axis_name="s")

    @functools.partial(
        pl.kernel,
        mesh=mesh,
        compiler_params=pltpu.CompilerParams(use_tc_tiling_on_sc=False),
        out_type=(jax.ShapeDtypeStruct((n_pad, hw), jnp.float32),
                  jax.ShapeDtypeStruct((n_pad, hw), jnp.float32)),
        scratch_types=[
            pltpu.VMEM((n_chunks, C), jnp.int32),
            pltpu.VMEM((n_chunks, C), jnp.int32),
        ]
        + [pltpu.VMEM((C, hw), jnp.float32) for _ in range(NBUF)]
        + [
            pltpu.VMEM_SHARED((n_pad, hw), jnp.float32),
        ]
        + [pltpu.SemaphoreType.DMA for _ in range(2 * NBUF)],
    )
    def k(ga_h, gb_h, src_h, dst_h, oa_h, ob_h, sall, dall, *rest):
        rows = rest[:NBUF]
        acc = rest[NBUF]
        gsem = rest[NBUF + 1:NBUF + 1 + NBUF]
        ssem = rest[NBUF + 1 + NBUF:]
        cid = lax.axis_index("c")
        sid = lax.axis_index("s")
        sl = pl.ds(sid * rps, rps)
        pltpu.sync_copy(src_h.at[sid], sall)
        pltpu.sync_copy(dst_h.at[sid], dall)

        def run(g_h, out_h):
            pltpu.sync_copy(g_h.at[sl], acc.at[sl])
            plsc.subcore_barrier()

            def gath(j, b):
                return pltpu.async_copy(g_h.at[sall.at[j]], rows[b],
                                        gsem[b])

            for j in range(PF):
                gath(j, j)

            def body(i, carry):
                j0 = i * NBUF
                for u in range(NBUF):
                    j = j0 + u
                    b = u
                    bn = (u + PF) % NBUF
                    pltpu.make_async_copy(g_h.at[sall.at[j]], rows[b],
                                          gsem[b]).wait()
                    pltpu.async_copy(rows[b], acc.at[dall.at[j]],
                                     ssem[b], add=True)
                    jp = j + PF - NBUF
                    @pl.when(jp >= 0)
                    def _():
                        pltpu.make_async_copy(rows[bn],
                                              acc.at[dall.at[jp]],
                                              ssem[bn]).wait()
                    @pl.when(j + PF < n_chunks)
                    def _():
                        gath(j + PF, bn)
                return carry

            lax.fori_loop(0, n_chunks // NBUF, body, 0, unroll=False)
            for j in range(n_chunks - (NBUF - PF), n_chunks):
                b = j % NBUF
                pltpu.make_async_copy(rows[b], acc.at[dall.at[j]],
                                      ssem[b]).wait()
            plsc.subcore_barrier()
            pltpu.sync_copy(acc.at[sl], out_h.at[sl])

        @pl.when(cid == 0)
        def _():
            run(ga_h, oa_h)

        @pl.when(cid == 1)
        def _():
            run(gb_h, ob_h)

    return k(ga, gb, src_l, dst_l)


def _sc_degree(dst_l, n_pad, n_chunks):
    rps = n_pad // NS
    ZB = 128
    assert rps % ZB == 0 and n_chunks % NBUF == 0
    mesh = plsc.VectorSubcoreMesh(core_axis_name="c", subcore_axis_name="s")

    @functools.partial(
        pl.kernel,
        mesh=mesh,
        compiler_params=pltpu.CompilerParams(use_tc_tiling_on_sc=False),
        out_type=jax.ShapeDtypeStruct((n_pad,), jnp.float32),
        scratch_types=[
            pltpu.VMEM((n_chunks, C), jnp.int32),
            pltpu.VMEM((ZB,), jnp.float32),
            pltpu.VMEM((ZB,), jnp.float32),
            pltpu.VMEM_SHARED((n_pad,), jnp.float32),
        ]
        + [pltpu.SemaphoreType.DMA for _ in range(NBUF)],
    )
    def k(dst_h, out_h, dall, ones_v, zero_v, acc, *sems):
        cid = lax.axis_index("c")
        sid = lax.axis_index("s")

        @pl.when(cid == 0)
        def _():
            for i in range(ZB // 16):
                ones_v[pl.ds(i * 16, 16)] = jnp.ones((16,), jnp.float32)
                zero_v[pl.ds(i * 16, 16)] = jnp.zeros((16,), jnp.float32)
            pltpu.sync_copy(dst_h.at[sid], dall)
            for t in range(rps // ZB):
                pltpu.sync_copy(zero_v,
                                acc.at[pl.ds(sid * rps + t * ZB, ZB)])
            plsc.subcore_barrier()
            ones_c = ones_v.at[pl.ds(0, C)]

            def body(i, carry):
                for u in range(NBUF):
                    j = i * NBUF + u
                    @pl.when(i > 0)
                    def _():
                        pltpu.make_async_copy(ones_c,
                                              acc.at[dall.at[j - NBUF]],
                                              sems[u]).wait()
                    pltpu.async_copy(ones_c, acc.at[dall.at[j]], sems[u],
                                     add=True)
                return carry

            lax.fori_loop(0, n_chunks // NBUF, body, 0, unroll=False)
            for u in range(NBUF):
                j = n_chunks - NBUF + u
                pltpu.make_async_copy(ones_c, acc.at[dall.at[j]],
                                      sems[u]).wait()
            plsc.subcore_barrier()
            pltpu.sync_copy(acc.at[pl.ds(sid * rps, rps)],
                            out_h.at[pl.ds(sid * rps, rps)])

    return k(dst_l)


def _dinv(deg_ref):
    return lax.rsqrt(deg_ref[0, :] + 1.0)[:, None]


def kernel(x, edge_index, batch, W1, b1, W2, b2, W3, b3, Wh, bh, Wo, bo):
    N, D = x.shape
    E = edge_index.shape[1]
    H = W1.shape[1]
    H2 = W2.shape[1]
    H3 = W3.shape[1]
    OUT = Wo.shape[1]

    n_pad = ((N + 2047) // 2048) * 2048
    pad_rows = n_pad - N

    Hh = H // 2
    bc = NS * C * NBUF
    e_pad = ((E + bc - 1) // bc) * bc
    n_chunks = e_pad // (NS * C)

    pi = jnp.arange(e_pad - E, dtype=jnp.int32)
    src_p = jnp.concatenate([edge_index[0], N + (pi % pad_rows)])
    dst_p = jnp.concatenate(
        [edge_index[1], N + ((pi * 7 + pad_rows // 2) % pad_rows)])
    src_l = src_p.reshape(NS, n_chunks, C)
    dst_l = dst_p.reshape(NS, n_chunks, C)
    x_pad = jnp.pad(x, ((0, pad_rows), (0, 0)))
    batch2d = jnp.pad(batch, (0, pad_rows), constant_values=64)[None, :]
    b1r, b2r, b3r = b1[None, :], b2[None, :], b3[None, :]
    bhr, bor = bh[None, :], bo[None, :]

    R = n_pad // NS
    T = NS

    deg = _sc_degree(dst_l, n_pad, n_chunks)
    deg2d = deg[None, :]

    def _b_body(x_ref, w_ref, deg_ref, oa_ref, ob_ref):
        h = jnp.dot(x_ref[...], w_ref[...], preferred_element_type=jnp.float32)
        g = h * _dinv(deg_ref)
        oa_ref[...] = g[:, :Hh]
        ob_ref[...] = g[:, Hh:]

    g1a, g1b = pl.pallas_call(
        _b_body,
        grid=(T,),
        in_specs=[pl.BlockSpec((R, D), lambda i: (i, 0)),
                  pl.BlockSpec((D, H), lambda i: (0, 0)),
                  pl.BlockSpec((1, R), lambda i: (0, i))],
        out_specs=[pl.BlockSpec((R, Hh), lambda i: (i, 0)),
                   pl.BlockSpec((R, Hh), lambda i: (i, 0))],
        out_shape=[jax.ShapeDtypeStruct((n_pad, Hh), jnp.float32),
                   jax.ShapeDtypeStruct((n_pad, Hh), jnp.float32)],
    )(x_pad, W1, deg2d)

    o1a, o1b = _sc_edge_scatter(g1a, g1b, src_l, dst_l, n_pad, Hh, n_chunks)

    def _c_body(oa_ref, ob_ref, deg_ref, b_ref, na_ref, nb_ref):
        dinv = _dinv(deg_ref)
        agg = dinv * jnp.concatenate([oa_ref[...], ob_ref[...]], axis=1)
        g2t = dinv * jnp.maximum(agg + b_ref[...], 0.0)
        na_ref[...] = g2t[:, :Hh]
        nb_ref[...] = g2t[:, Hh:]

    g2a, g2b = pl.pallas_call(
        _c_body,
        grid=(T,),
        in_specs=[pl.BlockSpec((R, Hh), lambda i: (i, 0)),
                  pl.BlockSpec((R, Hh), lambda i: (i, 0)),
                  pl.BlockSpec((1, R), lambda i: (0, i)),
                  pl.BlockSpec((1, H), lambda i: (0, 0))],
        out_specs=[pl.BlockSpec((R, Hh), lambda i: (i, 0)),
                   pl.BlockSpec((R, Hh), lambda i: (i, 0))],
        out_shape=[jax.ShapeDtypeStruct((n_pad, Hh), jnp.float32),
                   jax.ShapeDtypeStruct((n_pad, Hh), jnp.float32)],
    )(o1a, o1b, deg2d, b1r)

    o2a, o2b = _sc_edge_scatter(g2a, g2b, src_l, dst_l, n_pad, Hh, n_chunks)

    def _d_body(oa_ref, ob_ref, deg_ref, w_ref, b_ref, na_ref, nb_ref):
        dinv = _dinv(deg_ref)
        a = dinv * jnp.concatenate([oa_ref[...], ob_ref[...]], axis=1)
        c2 = jnp.maximum(
            jnp.dot(a, w_ref[...], preferred_element_type=jnp.float32)
            + b_ref[...], 0.0)
        g3t = dinv * c2
        na_ref[...] = g3t[:, :H]
        nb_ref[...] = g3t[:, H:]

    g3a, g3b = pl.pallas_call(
        _d_body,
        grid=(T,),
        in_specs=[pl.BlockSpec((R, Hh), lambda i: (i, 0)),
                  pl.BlockSpec((R, Hh), lambda i: (i, 0)),
                  pl.BlockSpec((1, R), lambda i: (0, i)),
                  pl.BlockSpec((H, H2), lambda i: (0, 0)),
                  pl.BlockSpec((1, H2), lambda i: (0, 0))],
        out_specs=[pl.BlockSpec((R, H), lambda i: (i, 0)),
                   pl.BlockSpec((R, H), lambda i: (i, 0))],
        out_shape=[jax.ShapeDtypeStruct((n_pad, H), jnp.float32),
                   jax.ShapeDtypeStruct((n_pad, H), jnp.float32)],
    )(o2a, o2b, deg2d, W2, b2r)

    o3a, o3b = _sc_edge_scatter(g3a, g3b, src_l, dst_l, n_pad, H, n_chunks)

    GR = 64

    def _e_body(oa_ref, ob_ref, deg_ref, w_ref, b_ref,
                batch_ref, sums_ref, cnts_ref):
        i = pl.program_id(0)

        @pl.when(i == 0)
        def _():
            sums_ref[...] = jnp.zeros_like(sums_ref)
            cnts_ref[...] = jnp.zeros_like(cnts_ref)

        dinv = _dinv(deg_ref)
        a = dinv * jnp.concatenate([oa_ref[...], ob_ref[...]], axis=1)
        c3 = jnp.maximum(
            jnp.dot(a, w_ref[...], preferred_element_type=jnp.float32)
            + b_ref[...], 0.0)
        onehot = (lax.broadcasted_iota(jnp.int32, (GR, R), 0)
                  == batch_ref[...]).astype(jnp.float32)
        sums_ref[...] += jnp.dot(onehot, c3,
                                 preferred_element_type=jnp.float32)
        cnts_ref[...] = cnts_ref[...] + jnp.sum(onehot, axis=1, keepdims=True)

    sums, cnts = pl.pallas_call(
        _e_body,
        grid=(T,),
        in_specs=[pl.BlockSpec((R, H), lambda i: (i, 0)),
                  pl.BlockSpec((R, H), lambda i: (i, 0)),
                  pl.BlockSpec((1, R), lambda i: (0, i)),
                  pl.BlockSpec((H2, H3), lambda i: (0, 0)),
                  pl.BlockSpec((1, H3), lambda i: (0, 0)),
                  pl.BlockSpec((1, R), lambda i: (0, i))],
        out_specs=[pl.BlockSpec((GR, H3), lambda i: (0, 0)),
                   pl.BlockSpec((GR, 128), lambda i: (0, 0))],
        out_shape=[jax.ShapeDtypeStruct((GR, H3), jnp.float32),
                   jax.ShapeDtypeStruct((GR, 128), jnp.float32)],
    )(o3a, o3b, deg2d, W3, b3r, batch2d)

    def _f_body(sums_ref, cnts_ref, wh_ref, bh_ref, wo_ref, bo_ref, o_ref):
        cnt = cnts_ref[:, 0:1]
        pooled = sums_ref[...] / jnp.maximum(cnt, 1.0)
        hid = jnp.maximum(
            jnp.dot(pooled, wh_ref[...], preferred_element_type=jnp.float32)
            + bh_ref[...], 0.0)
        logits = jnp.dot(hid, wo_ref[...],
                         preferred_element_type=jnp.float32) + bo_ref[...]
        o_ref[...] = jax.nn.sigmoid(logits)

    out = pl.pallas_call(
        _f_body,
        out_shape=jax.ShapeDtypeStruct((GR, OUT), jnp.float32),
    )(sums, cnts, Wh, bhr, Wo, bor)

    return out


# speedup vs baseline: 1.2034x; 1.0481x over previous
import functools

import jax
import jax.numpy as jnp
from jax import lax
from jax.experimental import pallas as pl
from jax.experimental.pallas import tpu as pltpu
from jax.experimental.pallas import tpu_sc as plsc

NC = 2
NS = 16
NW = NC * NS
C = 128
PF = 3


NBUF = 5


def _sc_edge_scatter(ga, gb, src_l, dst_l, n_pad, hw, n_chunks):
    rps = n_pad // NS
    assert n_chunks % NBUF == 0
    mesh = plsc.VectorSubcoreMesh(core_axis_name="c", subcore_axis_name="s")

    @functools.partial(
        pl.kernel,
        mesh=mesh,
        compiler_params=pltpu.CompilerParams(use_tc_tiling_on_sc=False),
        out_type=(jax.ShapeDtypeStruct((n_pad, hw), jnp.float32),
                  jax.ShapeDtypeStruct((n_pad, hw), jnp.float32)),
        scratch_types=[
            pltpu.VMEM((n_chunks * C,), jnp.int32),
            pltpu.VMEM((n_chunks * C,), jnp.int32),
        ]
        + [pltpu.VMEM((C, hw), jnp.float32) for _ in range(NBUF)]
        + [
            pltpu.VMEM_SHARED((n_pad, hw), jnp.float32),
        ]
        + [pltpu.SemaphoreType.DMA for _ in range(2 * NBUF)],
    )
    def k(ga_h, gb_h, src_h, dst_h, oa_h, ob_h, sall, dall, *rest):
        rows = rest[:NBUF]
        acc = rest[NBUF]
        gsem = rest[NBUF + 1:NBUF + 1 + NBUF]
        ssem = rest[NBUF + 1 + NBUF:]
        cid = lax.axis_index("c")
        sid = lax.axis_index("s")
        sl = pl.ds(sid * rps, rps)
        ew = n_chunks * C
        pltpu.sync_copy(src_h.at[pl.ds(sid * ew, ew)], sall)
        pltpu.sync_copy(dst_h.at[pl.ds(sid * ew, ew)], dall)

        def idx(ref, j):
            return ref.at[pl.ds(pl.multiple_of(j * C, C), C)]

        def run(g_h, out_h):
            pltpu.sync_copy(g_h.at[sl], acc.at[sl])
            plsc.subcore_barrier()

            def gath(j, b):
                return pltpu.async_copy(g_h.at[idx(sall, j)], rows[b],
                                        gsem[b])

            for j in range(PF):
                gath(j, j)

            def body(i, carry):
                j0 = i * NBUF
                for u in range(NBUF):
                    j = j0 + u
                    b = u
                    bn = (u + PF) % NBUF
                    pltpu.make_async_copy(g_h.at[idx(sall, j)], rows[b],
                                          gsem[b]).wait()
                    pltpu.async_copy(rows[b], acc.at[idx(dall, j)],
                                     ssem[b], add=True)
                    jp = j + PF - NBUF
                    @pl.when(jp >= 0)
                    def _():
                        pltpu.make_async_copy(rows[bn],
                                              acc.at[idx(dall, jp)],
                                              ssem[bn]).wait()
                    @pl.when(j + PF < n_chunks)
                    def _():
                        gath(j + PF, bn)
                return carry

            lax.fori_loop(0, n_chunks // NBUF, body, 0, unroll=False)
            for j in range(n_chunks - (NBUF - PF), n_chunks):
                b = j % NBUF
                pltpu.make_async_copy(rows[b], acc.at[idx(dall, j)],
                                      ssem[b]).wait()
            plsc.subcore_barrier()
            pltpu.sync_copy(acc.at[sl], out_h.at[sl])

        @pl.when(cid == 0)
        def _():
            run(ga_h, oa_h)

        @pl.when(cid == 1)
        def _():
            run(gb_h, ob_h)

    return k(ga, gb, src_l, dst_l)


def _sc_degree(dst_l, n_pad, n_chunks):
    rps = n_pad // NS
    ZB = 128
    assert rps % ZB == 0 and n_chunks % NBUF == 0
    mesh = plsc.VectorSubcoreMesh(core_axis_name="c", subcore_axis_name="s")

    @functools.partial(
        pl.kernel,
        mesh=mesh,
        compiler_params=pltpu.CompilerParams(use_tc_tiling_on_sc=False),
        out_type=jax.ShapeDtypeStruct((n_pad,), jnp.float32),
        scratch_types=[
            pltpu.VMEM((n_chunks * C,), jnp.int32),
            pltpu.VMEM((ZB,), jnp.float32),
            pltpu.VMEM((ZB,), jnp.float32),
            pltpu.VMEM_SHARED((n_pad,), jnp.float32),
        ]
        + [pltpu.SemaphoreType.DMA for _ in range(NBUF)],
    )
    def k(dst_h, out_h, dall, ones_v, zero_v, acc, *sems):
        cid = lax.axis_index("c")
        sid = lax.axis_index("s")

        @pl.when(cid == 0)
        def _():
            for i in range(ZB // 16):
                ones_v[pl.ds(i * 16, 16)] = jnp.ones((16,), jnp.float32)
                zero_v[pl.ds(i * 16, 16)] = jnp.zeros((16,), jnp.float32)
            ew = n_chunks * C
            pltpu.sync_copy(dst_h.at[pl.ds(sid * ew, ew)], dall)
            for t in range(rps // ZB):
                pltpu.sync_copy(zero_v,
                                acc.at[pl.ds(sid * rps + t * ZB, ZB)])
            plsc.subcore_barrier()
            ones_c = ones_v.at[pl.ds(0, C)]

            def idx(ref, j):
                return ref.at[pl.ds(pl.multiple_of(j * C, C), C)]

            def body(i, carry):
                for u in range(NBUF):
                    j = i * NBUF + u
                    @pl.when(i > 0)
                    def _():
                        pltpu.make_async_copy(ones_c,
                                              acc.at[idx(dall, j - NBUF)],
                                              sems[u]).wait()
                    pltpu.async_copy(ones_c, acc.at[idx(dall, j)], sems[u],
                                     add=True)
                return carry

            lax.fori_loop(0, n_chunks // NBUF, body, 0, unroll=False)
            for u in range(NBUF):
                j = n_chunks - NBUF + u
                pltpu.make_async_copy(ones_c, acc.at[idx(dall, j)],
                                      sems[u]).wait()
            plsc.subcore_barrier()
            pltpu.sync_copy(acc.at[pl.ds(sid * rps, rps)],
                            out_h.at[pl.ds(sid * rps, rps)])

    return k(dst_l)


def _dinv(deg_ref):
    return lax.rsqrt(deg_ref[0, :] + 1.0)[:, None]


def kernel(x, edge_index, batch, W1, b1, W2, b2, W3, b3, Wh, bh, Wo, bo):
    N, D = x.shape
    E = edge_index.shape[1]
    H = W1.shape[1]
    H2 = W2.shape[1]
    H3 = W3.shape[1]
    OUT = Wo.shape[1]

    n_pad = ((N + 2047) // 2048) * 2048
    pad_rows = n_pad - N

    Hh = H // 2
    bc = NS * C * NBUF
    e_pad = ((E + bc - 1) // bc) * bc
    n_chunks = e_pad // (NS * C)

    pi = jnp.arange(e_pad - E, dtype=jnp.int32)
    src_p = jnp.concatenate([edge_index[0], N + (pi % pad_rows)])
    dst_p = jnp.concatenate(
        [edge_index[1], N + ((pi * 7 + pad_rows // 2) % pad_rows)])
    src_l = src_p
    dst_l = dst_p
    x_pad = jnp.pad(x, ((0, pad_rows), (0, 0)))
    batch2d = jnp.pad(batch, (0, pad_rows), constant_values=64)[None, :]
    b1r, b2r, b3r = b1[None, :], b2[None, :], b3[None, :]
    bhr, bor = bh[None, :], bo[None, :]

    T = 8
    R = n_pad // T

    deg = _sc_degree(dst_l, n_pad, n_chunks)
    deg2d = deg[None, :]

    def _b_body(x_ref, w_ref, deg_ref, oa_ref, ob_ref):
        h = jnp.dot(x_ref[...], w_ref[...], preferred_element_type=jnp.float32)
        g = h * _dinv(deg_ref)
        oa_ref[...] = g[:, :Hh]
        ob_ref[...] = g[:, Hh:]

    g1a, g1b = pl.pallas_call(
        _b_body,
        grid=(T,),
        in_specs=[pl.BlockSpec((R, D), lambda i: (i, 0)),
                  pl.BlockSpec((D, H), lambda i: (0, 0)),
                  pl.BlockSpec((1, R), lambda i: (0, i))],
        out_specs=[pl.BlockSpec((R, Hh), lambda i: (i, 0)),
                   pl.BlockSpec((R, Hh), lambda i: (i, 0))],
        out_shape=[jax.ShapeDtypeStruct((n_pad, Hh), jnp.float32),
                   jax.ShapeDtypeStruct((n_pad, Hh), jnp.float32)],
    )(x_pad, W1, deg2d)

    o1a, o1b = _sc_edge_scatter(g1a, g1b, src_l, dst_l, n_pad, Hh, n_chunks)

    def _c_body(oa_ref, ob_ref, deg_ref, b_ref, na_ref, nb_ref):
        dinv = _dinv(deg_ref)
        agg = dinv * jnp.concatenate([oa_ref[...], ob_ref[...]], axis=1)
        g2t = dinv * jnp.maximum(agg + b_ref[...], 0.0)
        na_ref[...] = g2t[:, :Hh]
        nb_ref[...] = g2t[:, Hh:]

    g2a, g2b = pl.pallas_call(
        _c_body,
        grid=(T,),
        in_specs=[pl.BlockSpec((R, Hh), lambda i: (i, 0)),
                  pl.BlockSpec((R, Hh), lambda i: (i, 0)),
                  pl.BlockSpec((1, R), lambda i: (0, i)),
                  pl.BlockSpec((1, H), lambda i: (0, 0))],
        out_specs=[pl.BlockSpec((R, Hh), lambda i: (i, 0)),
                   pl.BlockSpec((R, Hh), lambda i: (i, 0))],
        out_shape=[jax.ShapeDtypeStruct((n_pad, Hh), jnp.float32),
                   jax.ShapeDtypeStruct((n_pad, Hh), jnp.float32)],
    )(o1a, o1b, deg2d, b1r)

    o2a, o2b = _sc_edge_scatter(g2a, g2b, src_l, dst_l, n_pad, Hh, n_chunks)

    def _d_body(oa_ref, ob_ref, deg_ref, w_ref, b_ref, na_ref, nb_ref):
        dinv = _dinv(deg_ref)
        a = dinv * jnp.concatenate([oa_ref[...], ob_ref[...]], axis=1)
        c2 = jnp.maximum(
            jnp.dot(a, w_ref[...], preferred_element_type=jnp.float32)
            + b_ref[...], 0.0)
        g3t = dinv * c2
        na_ref[...] = g3t[:, :H]
        nb_ref[...] = g3t[:, H:]

    g3a, g3b = pl.pallas_call(
        _d_body,
        grid=(T,),
        in_specs=[pl.BlockSpec((R, Hh), lambda i: (i, 0)),
                  pl.BlockSpec((R, Hh), lambda i: (i, 0)),
                  pl.BlockSpec((1, R), lambda i: (0, i)),
                  pl.BlockSpec((H, H2), lambda i: (0, 0)),
                  pl.BlockSpec((1, H2), lambda i: (0, 0))],
        out_specs=[pl.BlockSpec((R, H), lambda i: (i, 0)),
                   pl.BlockSpec((R, H), lambda i: (i, 0))],
        out_shape=[jax.ShapeDtypeStruct((n_pad, H), jnp.float32),
                   jax.ShapeDtypeStruct((n_pad, H), jnp.float32)],
    )(o2a, o2b, deg2d, W2, b2r)

    o3a, o3b = _sc_edge_scatter(g3a, g3b, src_l, dst_l, n_pad, H, n_chunks)

    GR = 64

    def _e_body(oa_ref, ob_ref, deg_ref, w_ref, b_ref,
                batch_ref, sums_ref, cnts_ref):
        i = pl.program_id(0)

        @pl.when(i == 0)
        def _():
            sums_ref[...] = jnp.zeros_like(sums_ref)
            cnts_ref[...] = jnp.zeros_like(cnts_ref)

        dinv = _dinv(deg_ref)
        a = dinv * jnp.concatenate([oa_ref[...], ob_ref[...]], axis=1)
        c3 = jnp.maximum(
            jnp.dot(a, w_ref[...], preferred_element_type=jnp.float32)
            + b_ref[...], 0.0)
        onehot = (lax.broadcasted_iota(jnp.int32, (GR, R), 0)
                  == batch_ref[...]).astype(jnp.float32)
        sums_ref[...] += jnp.dot(onehot, c3,
                                 preferred_element_type=jnp.float32)
        cnts_ref[...] = cnts_ref[...] + jnp.sum(onehot, axis=1, keepdims=True)

    sums, cnts = pl.pallas_call(
        _e_body,
        grid=(T,),
        in_specs=[pl.BlockSpec((R, H), lambda i: (i, 0)),
                  pl.BlockSpec((R, H), lambda i: (i, 0)),
                  pl.BlockSpec((1, R), lambda i: (0, i)),
                  pl.BlockSpec((H2, H3), lambda i: (0, 0)),
                  pl.BlockSpec((1, H3), lambda i: (0, 0)),
                  pl.BlockSpec((1, R), lambda i: (0, i))],
        out_specs=[pl.BlockSpec((GR, H3), lambda i: (0, 0)),
                   pl.BlockSpec((GR, 128), lambda i: (0, 0))],
        out_shape=[jax.ShapeDtypeStruct((GR, H3), jnp.float32),
                   jax.ShapeDtypeStruct((GR, 128), jnp.float32)],
    )(o3a, o3b, deg2d, W3, b3r, batch2d)

    def _f_body(sums_ref, cnts_ref, wh_ref, bh_ref, wo_ref, bo_ref, o_ref):
        cnt = cnts_ref[:, 0:1]
        pooled = sums_ref[...] / jnp.maximum(cnt, 1.0)
        hid = jnp.maximum(
            jnp.dot(pooled, wh_ref[...], preferred_element_type=jnp.float32)
            + bh_ref[...], 0.0)
        logits = jnp.dot(hid, wo_ref[...],
                         preferred_element_type=jnp.float32) + bo_ref[...]
        o_ref[...] = jax.nn.sigmoid(logits)

    out = pl.pallas_call(
        _f_body,
        out_shape=jax.ShapeDtypeStruct((GR, OUT), jnp.float32),
    )(sums, cnts, Wh, bhr, Wo, bor)

    return out
